# baseline (speedup 1.0000x reference)
import jax
import jax.numpy as jnp
from jax import lax
from jax.experimental import pallas as pl
from jax.experimental.pallas import tpu as pltpu
from jax.experimental.pallas import tpu_sc as plsc

B = 16384
D = 64
NC = 2
NS = 16
NW = NC * NS
BPW = B // NW
NCHUNK = 4
CHUNK = BPW // NCHUNK
NG = BPW // 16


def _rsqrt(x):
    i = plsc.bitcast(x, jnp.int32)
    i = jnp.int32(0x5F3759DF) - (i >> 1)
    y = plsc.bitcast(i, jnp.float32)
    for _ in range(3):
        y = y * (1.5 - 0.5 * x * y * y)
    return y


def _body(uT_h, mT_h, gT_h, yT_h, uidx_h, midx_h, gidx_h, yidx_h,
          ubias_h, mbias_h,
          out_h,
          uidx_v, midx_v, gidx_v, yidx_v,
          ubuf, mbuf, gtbl, ytbl, ub_v, mb_v, out_v, sem):
    wid = lax.axis_index("s") * NC + lax.axis_index("c")

    pltpu.sync_copy(uidx_h.at[wid], uidx_v)
    pltpu.sync_copy(midx_h.at[wid], midx_v)
    pltpu.sync_copy(gidx_h.at[wid], gidx_v)
    pltpu.sync_copy(yidx_h.at[wid], yidx_v)

    descs = [
        pltpu.async_copy(gT_h, gtbl, sem),
        pltpu.async_copy(yT_h, ytbl, sem),
    ]
    for j in range(NCHUNK):
        dst = pl.ds(j * CHUNK, CHUNK)
        descs.append(pltpu.async_copy(ubias_h.at[uidx_v.at[j]],
                                      ub_v.at[dst], sem))
        descs.append(pltpu.async_copy(mbias_h.at[midx_v.at[j]],
                                      mb_v.at[dst], sem))

    def plane(c, _):
        for j in range(NCHUNK):
            dst = pl.ds(j * CHUNK, CHUNK)
            pltpu.async_copy(uT_h.at[c].at[uidx_v.at[j]],
                             ubuf.at[c].at[dst], sem)
            pltpu.async_copy(mT_h.at[c].at[midx_v.at[j]],
                             mbuf.at[c].at[dst], sem)
        return 0

    lax.fori_loop(0, D, plane, 0)

    for dsc in descs:
        dsc.wait()
    pltpu.make_async_copy(uT_h.at[:, pl.ds(0, BPW)], ubuf, sem).wait()
    pltpu.make_async_copy(mT_h.at[:, pl.ds(0, BPW)], mbuf, sem).wait()

    def group(g, _):
        rows = g * 16 + lax.iota(jnp.int32, 16)
        base = g * 16
        giv = plsc.load_gather(gidx_v, [rows])
        yiv = plsc.load_gather(yidx_v, [rows])

        def col(c, carry):
            s_um, s_uu, s_mm = carry
            cv = jnp.broadcast_to(c, (16,))
            u = ubuf[c, pl.ds(base, 16)]
            mv = mbuf[c, pl.ds(base, 16)]
            gv = plsc.load_gather(gtbl, [cv, giv])
            yv = plsc.load_gather(ytbl, [cv, yiv])
            m = mv + gv + yv
            return (s_um + u * m, s_uu + u * u, s_mm + m * m)

        zeros = jnp.zeros((16,), jnp.float32)
        s_um, s_uu, s_mm = lax.fori_loop(
            0, D, col, (zeros, zeros, zeros), unroll=8)

        inv = _rsqrt(jnp.maximum(s_uu, 1e-16)) * _rsqrt(jnp.maximum(s_mm, 1e-16))
        ub = plsc.load_gather(ub_v, [rows])
        mb = plsc.load_gather(mb_v, [rows])
        pred = s_um * inv * 2.5 + 2.75 + ub + mb
        plsc.store_scatter(out_v, [rows], pred)
        return 0

    lax.fori_loop(0, NG, group, 0)

    base = pl.multiple_of(wid * BPW, BPW)
    pltpu.sync_copy(out_v, out_h.at[pl.ds(base, BPW)])


def kernel(user_idx, movie_idx, genre_idx, year_idx,
           user_embeds, movie_embeds, genre_embeds, year_embeds,
           user_biases, movie_biases):
    mesh = plsc.VectorSubcoreMesh(core_axis_name="c", subcore_axis_name="s",
                                  num_cores=NC, num_subcores=NS)
    f32 = jnp.float32
    i32 = jnp.int32
    k = pl.kernel(
        _body,
        out_type=jax.ShapeDtypeStruct((B,), f32),
        mesh=mesh,
        compiler_params=pltpu.CompilerParams(needs_layout_passes=False,
                                             use_tc_tiling_on_sc=False),
        scratch_types=[
            pltpu.VMEM((NCHUNK, CHUNK), i32),
            pltpu.VMEM((NCHUNK, CHUNK), i32),
            pltpu.VMEM((BPW,), i32),
            pltpu.VMEM((BPW,), i32),
            pltpu.VMEM((D, BPW), f32),
            pltpu.VMEM((D, BPW), f32),
            pltpu.VMEM((D, 20), f32),
            pltpu.VMEM((D, 100), f32),
            pltpu.VMEM((BPW,), f32),
            pltpu.VMEM((BPW,), f32),
            pltpu.VMEM((BPW,), f32),
            pltpu.SemaphoreType.DMA,
        ],
    )
    return k(user_embeds.T, movie_embeds.T, genre_embeds.T, year_embeds.T,
             user_idx.astype(i32).reshape(NW, NCHUNK, CHUNK),
             movie_idx.astype(i32).reshape(NW, NCHUNK, CHUNK),
             genre_idx.astype(i32).reshape(NW, BPW),
             year_idx.astype(i32).reshape(NW, BPW),
             user_biases.reshape(-1), movie_biases.reshape(-1))


# speedup vs baseline: 7.8923x; 7.8923x over previous
import jax
import jax.numpy as jnp
from jax import lax
from jax.experimental import pallas as pl
from jax.experimental.pallas import tpu as pltpu
from jax.experimental.pallas import tpu_sc as plsc

B = 16384
D = 64
DP = 128
NC = 2
NS = 16
NW = NC * NS
BPW = B // NW
HALF = BPW // 2
CHUNK = 128
NCHUNK = BPW // CHUNK


def _rsqrt(x):
    i = plsc.bitcast(x, jnp.int32)
    i = jnp.int32(0x5F3759DF) - (i >> 1)
    y = plsc.bitcast(i, jnp.float32)
    for _ in range(3):
        y = y * (1.5 - 0.5 * x * y * y)
    return y


def _body(u_h, m_h, g_h, y_h, uidx_h, midx_h, gidx_h, yidx_h,
          ubias_h, mbias_h,
          out_h,
          uidx_v, midx_v, gidx_v, yidx_v,
          urows, mrows, gtbl, ytbl, ub_v, mb_v, out_v, sem):
    wid = lax.axis_index("s") * NC + lax.axis_index("c")

    pltpu.sync_copy(uidx_h.at[wid], uidx_v)
    pltpu.sync_copy(midx_h.at[wid], midx_v)
    pltpu.sync_copy(gidx_h.at[wid], gidx_v)
    pltpu.sync_copy(yidx_h.at[wid], yidx_v)

    tdescs = [
        pltpu.async_copy(g_h, gtbl, sem),
        pltpu.async_copy(y_h, ytbl, sem),
    ]
    for j in range(NCHUNK):
        dst = pl.ds(j * CHUNK, CHUNK)
        tdescs.append(pltpu.async_copy(ubias_h.at[uidx_v.at[j]],
                                       ub_v.at[dst], sem))
        tdescs.append(pltpu.async_copy(mbias_h.at[midx_v.at[j]],
                                       mb_v.at[dst], sem))

    def fire_chunk(q):
        b = q % 2
        return [
            pltpu.async_copy(u_h.at[uidx_v.at[q]], urows.at[b], sem),
            pltpu.async_copy(m_h.at[midx_v.at[q]], mrows.at[b], sem),
        ]

    descs = [fire_chunk(0), fire_chunk(1)]
    for dsc in tdescs:
        dsc.wait()

    def run_chunk(q):
        b = q % 2
        for dsc in descs[q]:
            dsc.wait()

        def group(g, _):
            rows = g * 16 + lax.iota(jnp.int32, 16)
            grows = q * CHUNK + g * 16 + lax.iota(jnp.int32, 16)
            giv = plsc.load_gather(gidx_v, [grows])
            yiv = plsc.load_gather(yidx_v, [grows])

            def col(c, carry):
                s_um, s_uu, s_mm = carry
                cv = jnp.broadcast_to(c, (16,))
                u = plsc.load_gather(urows.at[b], [rows, cv])
                mv = plsc.load_gather(mrows.at[b], [rows, cv])
                gv = plsc.load_gather(gtbl, [giv, cv])
                yv = plsc.load_gather(ytbl, [yiv, cv])
                m = mv + gv + yv
                return (s_um + u * m, s_uu + u * u, s_mm + m * m)

            zeros = jnp.zeros((16,), jnp.float32)
            s_um, s_uu, s_mm = lax.fori_loop(
                0, D, col, (zeros, zeros, zeros), unroll=8)

            inv = (_rsqrt(jnp.maximum(s_uu, 1e-16))
                   * _rsqrt(jnp.maximum(s_mm, 1e-16)))
            ub = plsc.load_gather(ub_v, [grows])
            mb = plsc.load_gather(mb_v, [grows])
            pred = s_um * inv * 2.5 + 2.75 + ub + mb
            plsc.store_scatter(out_v, [grows], pred)
            return 0

        lax.fori_loop(0, CHUNK // 16, group, 0)

    for q in range(NCHUNK):
        run_chunk(q)
        if q + 2 < NCHUNK:
            descs.append(fire_chunk(q + 2))

    base = pl.multiple_of(wid * BPW, BPW)
    pltpu.sync_copy(out_v, out_h.at[pl.ds(base, BPW)])


def kernel(user_idx, movie_idx, genre_idx, year_idx,
           user_embeds, movie_embeds, genre_embeds, year_embeds,
           user_biases, movie_biases):
    mesh = plsc.VectorSubcoreMesh(core_axis_name="c", subcore_axis_name="s",
                                  num_cores=NC, num_subcores=NS)
    f32 = jnp.float32
    i32 = jnp.int32
    k = pl.kernel(
        _body,
        out_type=jax.ShapeDtypeStruct((B,), f32),
        mesh=mesh,
        compiler_params=pltpu.CompilerParams(needs_layout_passes=False,
                                             use_tc_tiling_on_sc=False),
        scratch_types=[
            pltpu.VMEM((NCHUNK, CHUNK), i32),
            pltpu.VMEM((NCHUNK, CHUNK), i32),
            pltpu.VMEM((BPW,), i32),
            pltpu.VMEM((BPW,), i32),
            pltpu.VMEM((2, CHUNK, DP), f32),
            pltpu.VMEM((2, CHUNK, DP), f32),
            pltpu.VMEM((20, D), f32),
            pltpu.VMEM((100, D), f32),
            pltpu.VMEM((BPW,), f32),
            pltpu.VMEM((BPW,), f32),
            pltpu.VMEM((BPW,), f32),
            pltpu.SemaphoreType.DMA,
        ],
    )
    upad = jnp.pad(user_embeds, ((0, 0), (0, DP - D)))
    mpad = jnp.pad(movie_embeds, ((0, 0), (0, DP - D)))
    return k(upad, mpad, genre_embeds, year_embeds,
             user_idx.astype(i32).reshape(NW, NCHUNK, CHUNK),
             movie_idx.astype(i32).reshape(NW, NCHUNK, CHUNK),
             genre_idx.astype(i32).reshape(NW, BPW),
             year_idx.astype(i32).reshape(NW, BPW),
             user_biases.reshape(-1), movie_biases.reshape(-1))


# speedup vs baseline: 9.1177x; 1.1553x over previous
import jax
import jax.numpy as jnp
from jax import lax
from jax.experimental import pallas as pl
from jax.experimental.pallas import tpu as pltpu
from jax.experimental.pallas import tpu_sc as plsc

B = 16384
D = 64
NC = 2
NS = 16
NW = NC * NS
BPW = B // NW
NCHUNK = 4
CHUNK = BPW // NCHUNK
NG = BPW // 16

CHK = 65536

U_ROWS, U_FULL, U_TAIL = 1000000, 999936, 64
U_NFULL = U_FULL // CHK
U_REM = U_FULL - U_NFULL * CHK
U_S = 1000448
U_EDGE = U_S - U_NFULL * CHK

M_ROWS, M_FULL_C, M_TAIL = 100000, 99968, 32
M_NFULL = M_FULL_C // CHK
M_REM = M_FULL_C - M_NFULL * CHK
M_S = 100352
M_EDGE = M_S - M_NFULL * CHK


def _detile_body(rem, tail_w, stride, src_ref, tail_ref, out_ref,
                 buf, tbuf, ebuf, sem):
    a = pl.program_id(0)
    k = pl.program_id(1)
    nfull = pl.num_programs(1) - 1

    @pl.when(k < nfull)
    def _main():
        pltpu.sync_copy(src_ref.at[pl.ds(a * 8, 8), pl.ds(k * CHK, CHK)], buf)
        descs = []
        for i in range(8):
            dst = pl.ds((a * 8 + i) * stride + k * CHK, CHK)
            descs.append(pltpu.async_copy(buf.at[i], out_ref.at[dst], sem))
        for dsc in descs:
            dsc.wait()

    @pl.when(k == nfull)
    def _edge():
        pltpu.sync_copy(src_ref.at[pl.ds(a * 8, 8), pl.ds(nfull * CHK, rem)],
                        ebuf.at[:, pl.ds(0, rem)])
        pltpu.sync_copy(tail_ref.at[pl.ds(a * 8, 8)], tbuf)
        ebuf[:, pl.ds(rem, tail_w)] = tbuf[...]
        edge = ebuf.shape[1]
        descs = []
        for i in range(8):
            dst = pl.ds((a * 8 + i) * stride + nfull * CHK, edge)
            descs.append(pltpu.async_copy(ebuf.at[i], out_ref.at[dst], sem))
        for dsc in descs:
            dsc.wait()


def _detile(table_t, tail_t, nfull, rem, tail_w, stride):
    import functools
    body = functools.partial(_detile_body, rem, tail_w, stride)
    edge = stride - nfull * CHK
    return pl.pallas_call(
        body,
        grid=(8, nfull + 1),
        in_specs=[pl.BlockSpec(memory_space=pl.ANY),
                  pl.BlockSpec(memory_space=pl.ANY)],
        out_specs=pl.BlockSpec(memory_space=pl.ANY),
        out_shape=jax.ShapeDtypeStruct((D * stride,), jnp.float32),
        scratch_shapes=[
            pltpu.VMEM((8, CHK), jnp.float32),
            pltpu.VMEM((8, tail_w), jnp.float32),
            pltpu.VMEM((8, edge), jnp.float32),
            pltpu.SemaphoreType.DMA,
        ],
    )(table_t, tail_t)


def _rsqrt(x):
    i = plsc.bitcast(x, jnp.int32)
    i = jnp.int32(0x5F3759DF) - (i >> 1)
    y = plsc.bitcast(i, jnp.float32)
    for _ in range(3):
        y = y * (1.5 - 0.5 * x * y * y)
    return y


def _body(uT_h, mT_h, gT_h, yT_h, uidx_h, midx_h, gidx_h, yidx_h,
          ubias_h, mbias_h,
          out_h,
          uidx_v, midx_v, gidx_v, yidx_v,
          ubuf, mbuf, gtbl, ytbl, ub_v, mb_v, out_v, sem):
    wid = lax.axis_index("s") * NC + lax.axis_index("c")

    pltpu.sync_copy(uidx_h.at[wid], uidx_v)
    pltpu.sync_copy(midx_h.at[wid], midx_v)
    pltpu.sync_copy(gidx_h.at[wid], gidx_v)
    pltpu.sync_copy(yidx_h.at[wid], yidx_v)

    descs = [
        pltpu.async_copy(gT_h, gtbl, sem),
        pltpu.async_copy(yT_h, ytbl, sem),
    ]
    for j in range(NCHUNK):
        dst = pl.ds(j * CHUNK, CHUNK)
        descs.append(pltpu.async_copy(ubias_h.at[uidx_v.at[j]],
                                      ub_v.at[dst], sem))
        descs.append(pltpu.async_copy(mbias_h.at[midx_v.at[j]],
                                      mb_v.at[dst], sem))

    def plane(c, _):
        for j in range(NCHUNK):
            dst = pl.ds(j * CHUNK, CHUNK)
            pltpu.async_copy(uT_h.at[c].at[uidx_v.at[j]],
                             ubuf.at[c].at[dst], sem)
            pltpu.async_copy(mT_h.at[c].at[midx_v.at[j]],
                             mbuf.at[c].at[dst], sem)
        return 0

    lax.fori_loop(0, D, plane, 0)

    for dsc in descs:
        dsc.wait()
    pltpu.make_async_copy(uT_h.at[:, pl.ds(0, BPW)], ubuf, sem).wait()
    pltpu.make_async_copy(mT_h.at[:, pl.ds(0, BPW)], mbuf, sem).wait()

    def group(g, _):
        rows = g * 16 + lax.iota(jnp.int32, 16)
        base = g * 16
        giv = plsc.load_gather(gidx_v, [rows])
        yiv = plsc.load_gather(yidx_v, [rows])

        def col(c, carry):
            s_um, s_uu, s_mm = carry
            cv = jnp.broadcast_to(c, (16,))
            u = ubuf[c, pl.ds(base, 16)]
            mv = mbuf[c, pl.ds(base, 16)]
            gv = plsc.load_gather(gtbl, [cv, giv])
            yv = plsc.load_gather(ytbl, [cv, yiv])
            m = mv + gv + yv
            return (s_um + u * m, s_uu + u * u, s_mm + m * m)

        zeros = jnp.zeros((16,), jnp.float32)
        s_um, s_uu, s_mm = lax.fori_loop(
            0, D, col, (zeros, zeros, zeros), unroll=8)

        inv = _rsqrt(jnp.maximum(s_uu, 1e-16)) * _rsqrt(jnp.maximum(s_mm, 1e-16))
        ub = plsc.load_gather(ub_v, [rows])
        mb = plsc.load_gather(mb_v, [rows])
        pred = s_um * inv * 2.5 + 2.75 + ub + mb
        plsc.store_scatter(out_v, [rows], pred)
        return 0

    lax.fori_loop(0, NG, group, 0)

    base = pl.multiple_of(wid * BPW, BPW)
    pltpu.sync_copy(out_v, out_h.at[pl.ds(base, BPW)])


def kernel(user_idx, movie_idx, genre_idx, year_idx,
           user_embeds, movie_embeds, genre_embeds, year_embeds,
           user_biases, movie_biases):
    mesh = plsc.VectorSubcoreMesh(core_axis_name="c", subcore_axis_name="s",
                                  num_cores=NC, num_subcores=NS)
    f32 = jnp.float32
    i32 = jnp.int32
    k = pl.kernel(
        _body,
        out_type=jax.ShapeDtypeStruct((B,), f32),
        mesh=mesh,
        compiler_params=pltpu.CompilerParams(needs_layout_passes=False,
                                             use_tc_tiling_on_sc=False),
        scratch_types=[
            pltpu.VMEM((NCHUNK, CHUNK), i32),
            pltpu.VMEM((NCHUNK, CHUNK), i32),
            pltpu.VMEM((BPW,), i32),
            pltpu.VMEM((BPW,), i32),
            pltpu.VMEM((D, BPW), f32),
            pltpu.VMEM((D, BPW), f32),
            pltpu.VMEM((D, 20), f32),
            pltpu.VMEM((D, 100), f32),
            pltpu.VMEM((BPW,), f32),
            pltpu.VMEM((BPW,), f32),
            pltpu.VMEM((BPW,), f32),
            pltpu.SemaphoreType.DMA,
        ],
    )
    uLin = _detile(user_embeds.T, user_embeds[U_FULL:].T,
                   U_NFULL, U_REM, U_TAIL, U_S)
    mLin = _detile(movie_embeds.T, movie_embeds[M_FULL_C:].T,
                   M_NFULL, M_REM, M_TAIL, M_S)
    return k(uLin.reshape(D, U_S), mLin.reshape(D, M_S),
             genre_embeds.T, year_embeds.T,
             user_idx.astype(i32).reshape(NW, NCHUNK, CHUNK),
             movie_idx.astype(i32).reshape(NW, NCHUNK, CHUNK),
             genre_idx.astype(i32).reshape(NW, BPW),
             year_idx.astype(i32).reshape(NW, BPW),
             user_biases.reshape(-1), movie_biases.reshape(-1))


# speedup vs baseline: 12.1607x; 1.3337x over previous
import jax
import jax.numpy as jnp
from jax import lax
from jax.experimental import pallas as pl
from jax.experimental.pallas import tpu as pltpu
from jax.experimental.pallas import tpu_sc as plsc

B = 16384
D = 64
NC = 2
NS = 16
NW = NC * NS
BPW = B // NW
NCHUNK = 4
CHUNK = BPW // NCHUNK
NG = BPW // 16

CHK = 65536

U_ROWS, U_FULL, U_TAIL = 1000000, 999936, 64
U_NFULL = U_FULL // CHK
U_REM = U_FULL - U_NFULL * CHK
U_S = 1000448
U_EDGE = U_S - U_NFULL * CHK

M_ROWS, M_FULL_C, M_TAIL = 100000, 99968, 32
M_NFULL = M_FULL_C // CHK
M_REM = M_FULL_C - M_NFULL * CHK
M_S = 100352
M_EDGE = M_S - M_NFULL * CHK


def _detile_body(rem, tail_w, stride, src_ref, tail_ref, out_ref,
                 buf, tbuf, ebuf, sem_in, sem_out):
    a = pl.program_id(0)
    k = pl.program_id(1)
    nfull = pl.num_programs(1) - 1
    edge = ebuf.shape[1]

    def start_main(kk, b):
        pltpu.make_async_copy(
            src_ref.at[pl.ds(a * 8, 8), pl.ds(kk * CHK, CHK)],
            buf.at[b], sem_in).start()

    @pl.when(k == 0)
    def _prime():
        start_main(0, 0)

    @pl.when(k >= 1)
    def _drain_prev():
        for i in range(8):
            pltpu.make_async_copy(buf.at[0].at[i], out_ref.at[pl.ds(0, CHK)],
                                  sem_out).wait()

    @pl.when(k + 1 < nfull)
    def _prefetch_main():
        start_main(k + 1, (k + 1) % 2)

    @pl.when(k + 1 == nfull)
    def _prefetch_edge():
        pltpu.make_async_copy(
            src_ref.at[pl.ds(a * 8, 8), pl.ds(nfull * CHK, rem)],
            ebuf.at[:, pl.ds(0, rem)], sem_in).start()
        pltpu.make_async_copy(tail_ref.at[pl.ds(a * 8, 8)], tbuf,
                              sem_in).start()

    @pl.when(k < nfull)
    def _main():
        b = k % 2
        pltpu.make_async_copy(
            src_ref.at[pl.ds(a * 8, 8), pl.ds(k * CHK, CHK)],
            buf.at[b], sem_in).wait()
        for i in range(8):
            dst = pl.ds((a * 8 + i) * stride + k * CHK, CHK)
            pltpu.make_async_copy(buf.at[b].at[i], out_ref.at[dst],
                                  sem_out).start()

    @pl.when(k == nfull)
    def _edge():
        pltpu.make_async_copy(
            src_ref.at[pl.ds(a * 8, 8), pl.ds(nfull * CHK, rem)],
            ebuf.at[:, pl.ds(0, rem)], sem_in).wait()
        pltpu.make_async_copy(tail_ref.at[pl.ds(a * 8, 8)], tbuf,
                              sem_in).wait()
        ebuf[:, pl.ds(rem, tail_w)] = tbuf[...]
        descs = []
        for i in range(8):
            dst = pl.ds((a * 8 + i) * stride + nfull * CHK, edge)
            descs.append(pltpu.async_copy(ebuf.at[i], out_ref.at[dst],
                                          sem_out))
        for dsc in descs:
            dsc.wait()


def _detile(table_t, tail_t, nfull, rem, tail_w, stride):
    import functools
    body = functools.partial(_detile_body, rem, tail_w, stride)
    edge = stride - nfull * CHK
    return pl.pallas_call(
        body,
        grid=(8, nfull + 1),
        in_specs=[pl.BlockSpec(memory_space=pl.ANY),
                  pl.BlockSpec(memory_space=pl.ANY)],
        out_specs=pl.BlockSpec(memory_space=pl.ANY),
        out_shape=jax.ShapeDtypeStruct((D * stride,), jnp.float32),
        scratch_shapes=[
            pltpu.VMEM((2, 8, CHK), jnp.float32),
            pltpu.VMEM((8, tail_w), jnp.float32),
            pltpu.VMEM((8, edge), jnp.float32),
            pltpu.SemaphoreType.DMA,
            pltpu.SemaphoreType.DMA,
        ],
    )(table_t, tail_t)


def _rsqrt(x):
    i = plsc.bitcast(x, jnp.int32)
    i = jnp.int32(0x5F3759DF) - (i >> 1)
    y = plsc.bitcast(i, jnp.float32)
    for _ in range(3):
        y = y * (1.5 - 0.5 * x * y * y)
    return y


def _body(uT_h, mT_h, gT_h, yT_h, uidx_h, midx_h, gidx_h, yidx_h,
          ubias_h, mbias_h,
          out_h,
          uidx_v, midx_v, gidx_v, yidx_v,
          ubuf, mbuf, gtbl, ytbl, ub_v, mb_v, out_v, sem):
    wid = lax.axis_index("s") * NC + lax.axis_index("c")

    pltpu.sync_copy(uidx_h.at[wid], uidx_v)
    pltpu.sync_copy(midx_h.at[wid], midx_v)
    pltpu.sync_copy(gidx_h.at[wid], gidx_v)
    pltpu.sync_copy(yidx_h.at[wid], yidx_v)

    descs = [
        pltpu.async_copy(gT_h, gtbl, sem),
        pltpu.async_copy(yT_h, ytbl, sem),
    ]
    for j in range(NCHUNK):
        dst = pl.ds(j * CHUNK, CHUNK)
        descs.append(pltpu.async_copy(ubias_h.at[uidx_v.at[j]],
                                      ub_v.at[dst], sem))
        descs.append(pltpu.async_copy(mbias_h.at[midx_v.at[j]],
                                      mb_v.at[dst], sem))

    def plane(c, _):
        for j in range(NCHUNK):
            dst = pl.ds(j * CHUNK, CHUNK)
            pltpu.async_copy(uT_h.at[c].at[uidx_v.at[j]],
                             ubuf.at[c].at[dst], sem)
            pltpu.async_copy(mT_h.at[c].at[midx_v.at[j]],
                             mbuf.at[c].at[dst], sem)
        return 0

    lax.fori_loop(0, D, plane, 0)

    for dsc in descs:
        dsc.wait()
    pltpu.make_async_copy(uT_h.at[:, pl.ds(0, BPW)], ubuf, sem).wait()
    pltpu.make_async_copy(mT_h.at[:, pl.ds(0, BPW)], mbuf, sem).wait()

    def group(g, _):
        rows = g * 16 + lax.iota(jnp.int32, 16)
        base = g * 16
        giv = plsc.load_gather(gidx_v, [rows])
        yiv = plsc.load_gather(yidx_v, [rows])

        def col(c, carry):
            s_um, s_uu, s_mm = carry
            cv = jnp.broadcast_to(c, (16,))
            u = ubuf[c, pl.ds(base, 16)]
            mv = mbuf[c, pl.ds(base, 16)]
            gv = plsc.load_gather(gtbl, [cv, giv])
            yv = plsc.load_gather(ytbl, [cv, yiv])
            m = mv + gv + yv
            return (s_um + u * m, s_uu + u * u, s_mm + m * m)

        zeros = jnp.zeros((16,), jnp.float32)
        s_um, s_uu, s_mm = lax.fori_loop(
            0, D, col, (zeros, zeros, zeros), unroll=8)

        inv = _rsqrt(jnp.maximum(s_uu, 1e-16)) * _rsqrt(jnp.maximum(s_mm, 1e-16))
        ub = plsc.load_gather(ub_v, [rows])
        mb = plsc.load_gather(mb_v, [rows])
        pred = s_um * inv * 2.5 + 2.75 + ub + mb
        plsc.store_scatter(out_v, [rows], pred)
        return 0

    lax.fori_loop(0, NG, group, 0)

    base = pl.multiple_of(wid * BPW, BPW)
    pltpu.sync_copy(out_v, out_h.at[pl.ds(base, BPW)])


def kernel(user_idx, movie_idx, genre_idx, year_idx,
           user_embeds, movie_embeds, genre_embeds, year_embeds,
           user_biases, movie_biases):
    mesh = plsc.VectorSubcoreMesh(core_axis_name="c", subcore_axis_name="s",
                                  num_cores=NC, num_subcores=NS)
    f32 = jnp.float32
    i32 = jnp.int32
    k = pl.kernel(
        _body,
        out_type=jax.ShapeDtypeStruct((B,), f32),
        mesh=mesh,
        compiler_params=pltpu.CompilerParams(needs_layout_passes=False,
                                             use_tc_tiling_on_sc=False),
        scratch_types=[
            pltpu.VMEM((NCHUNK, CHUNK), i32),
            pltpu.VMEM((NCHUNK, CHUNK), i32),
            pltpu.VMEM((BPW,), i32),
            pltpu.VMEM((BPW,), i32),
            pltpu.VMEM((D, BPW), f32),
            pltpu.VMEM((D, BPW), f32),
            pltpu.VMEM((D, 20), f32),
            pltpu.VMEM((D, 100), f32),
            pltpu.VMEM((BPW,), f32),
            pltpu.VMEM((BPW,), f32),
            pltpu.VMEM((BPW,), f32),
            pltpu.SemaphoreType.DMA,
        ],
    )
    uLin = _detile(user_embeds.T, user_embeds[U_FULL:].T,
                   U_NFULL, U_REM, U_TAIL, U_S)
    mLin = _detile(movie_embeds.T, movie_embeds[M_FULL_C:].T,
                   M_NFULL, M_REM, M_TAIL, M_S)
    return k(uLin.reshape(D, U_S), mLin.reshape(D, M_S),
             genre_embeds.T, year_embeds.T,
             user_idx.astype(i32).reshape(NW, NCHUNK, CHUNK),
             movie_idx.astype(i32).reshape(NW, NCHUNK, CHUNK),
             genre_idx.astype(i32).reshape(NW, BPW),
             year_idx.astype(i32).reshape(NW, BPW),
             user_biases.reshape(-1), movie_biases.reshape(-1))


# speedup vs baseline: 13.7131x; 1.1277x over previous
import jax
import jax.numpy as jnp
from jax import lax
from jax.experimental import pallas as pl
from jax.experimental.pallas import tpu as pltpu
from jax.experimental.pallas import tpu_sc as plsc

B = 16384
D = 64
NC = 2
NS = 16
NW = NC * NS
BPW = B // NW
NCHUNK = 4
CHUNK = BPW // NCHUNK
NG = BPW // 16

CHK = 65536

U_ROWS, U_FULL, U_TAIL = 1000000, 999936, 64
U_NFULL = U_FULL // CHK
U_REM = U_FULL - U_NFULL * CHK
U_S = 1000448
U_EDGE = U_S - U_NFULL * CHK

M_ROWS, M_FULL_C, M_TAIL = 100000, 99968, 32
M_NFULL = M_FULL_C // CHK
M_REM = M_FULL_C - M_NFULL * CHK
M_S = 100352
M_EDGE = M_S - M_NFULL * CHK


def _detile_body(rem, tail_w, stride, src_ref, tail_ref, out_ref,
                 buf, tbuf, ebuf, sem_in, sem_out):
    a = pl.program_id(0)
    k = pl.program_id(1)
    nfull = pl.num_programs(1) - 1
    edge = ebuf.shape[1]

    def start_main(kk, b):
        pltpu.make_async_copy(
            src_ref.at[pl.ds(a * 8, 8), pl.ds(kk * CHK, CHK)],
            buf.at[b], sem_in).start()

    @pl.when(k == 0)
    def _prime():
        start_main(0, 0)
        if nfull > 1:
            start_main(1, 1)

    @pl.when(k >= 1)
    def _drain_prev():
        for i in range(8):
            pltpu.make_async_copy(buf.at[0].at[i], out_ref.at[pl.ds(0, CHK)],
                                  sem_out).wait()

    @pl.when(k + 2 < nfull)
    def _prefetch_main():
        start_main(k + 2, (k + 2) % 3)

    @pl.when(k + 1 == nfull)
    def _prefetch_edge():
        pltpu.make_async_copy(
            src_ref.at[pl.ds(a * 8, 8), pl.ds(nfull * CHK, rem)],
            ebuf.at[:, pl.ds(0, rem)], sem_in).start()
        pltpu.make_async_copy(tail_ref.at[pl.ds(a * 8, 8)], tbuf,
                              sem_in).start()

    @pl.when(k < nfull)
    def _main():
        b = k % 3
        pltpu.make_async_copy(
            src_ref.at[pl.ds(a * 8, 8), pl.ds(k * CHK, CHK)],
            buf.at[b], sem_in).wait()
        for i in range(8):
            dst = pl.ds((a * 8 + i) * stride + k * CHK, CHK)
            pltpu.make_async_copy(buf.at[b].at[i], out_ref.at[dst],
                                  sem_out).start()

    @pl.when(k == nfull)
    def _edge():
        pltpu.make_async_copy(
            src_ref.at[pl.ds(a * 8, 8), pl.ds(nfull * CHK, rem)],
            ebuf.at[:, pl.ds(0, rem)], sem_in).wait()
        pltpu.make_async_copy(tail_ref.at[pl.ds(a * 8, 8)], tbuf,
                              sem_in).wait()
        ebuf[:, pl.ds(rem, tail_w)] = tbuf[...]
        descs = []
        for i in range(8):
            dst = pl.ds((a * 8 + i) * stride + nfull * CHK, edge)
            descs.append(pltpu.async_copy(ebuf.at[i], out_ref.at[dst],
                                          sem_out))
        for dsc in descs:
            dsc.wait()


def _detile(table_t, tail_t, nfull, rem, tail_w, stride):
    import functools
    body = functools.partial(_detile_body, rem, tail_w, stride)
    edge = stride - nfull * CHK
    return pl.pallas_call(
        body,
        grid=(8, nfull + 1),
        in_specs=[pl.BlockSpec(memory_space=pl.ANY),
                  pl.BlockSpec(memory_space=pl.ANY)],
        out_specs=pl.BlockSpec(memory_space=pl.ANY),
        out_shape=jax.ShapeDtypeStruct((D * stride,), jnp.float32),
        scratch_shapes=[
            pltpu.VMEM((3, 8, CHK), jnp.float32),
            pltpu.VMEM((8, tail_w), jnp.float32),
            pltpu.VMEM((8, edge), jnp.float32),
            pltpu.SemaphoreType.DMA,
            pltpu.SemaphoreType.DMA,
        ],
    )(table_t, tail_t)


def _rsqrt(x):
    i = plsc.bitcast(x, jnp.int32)
    i = jnp.int32(0x5F3759DF) - (i >> 1)
    y = plsc.bitcast(i, jnp.float32)
    for _ in range(3):
        y = y * (1.5 - 0.5 * x * y * y)
    return y


def _body(uT_h, mT_h, gT_h, yT_h, uidx_h, midx_h, gidx_h, yidx_h,
          ubias_h, mbias_h,
          out_h,
          uidx_v, midx_v, gidx_v, yidx_v,
          ubuf, mbuf, gtbl, ytbl, ub_v, mb_v, out_v, sem):
    wid = lax.axis_index("s") * NC + lax.axis_index("c")

    pltpu.sync_copy(uidx_h.at[wid], uidx_v)
    pltpu.sync_copy(midx_h.at[wid], midx_v)
    pltpu.sync_copy(gidx_h.at[wid], gidx_v)
    pltpu.sync_copy(yidx_h.at[wid], yidx_v)

    descs = [
        pltpu.async_copy(gT_h, gtbl, sem),
        pltpu.async_copy(yT_h, ytbl, sem),
    ]
    for j in range(NCHUNK):
        dst = pl.ds(j * CHUNK, CHUNK)
        descs.append(pltpu.async_copy(ubias_h.at[0].at[uidx_v.at[j]],
                                      ub_v.at[dst], sem))
        descs.append(pltpu.async_copy(mbias_h.at[0].at[midx_v.at[j]],
                                      mb_v.at[dst], sem))

    def plane(c, _):
        for j in range(NCHUNK):
            dst = pl.ds(j * CHUNK, CHUNK)
            pltpu.async_copy(uT_h.at[c].at[uidx_v.at[j]],
                             ubuf.at[c].at[dst], sem)
            pltpu.async_copy(mT_h.at[c].at[midx_v.at[j]],
                             mbuf.at[c].at[dst], sem)
        return 0

    lax.fori_loop(0, D, plane, 0)

    for dsc in descs:
        dsc.wait()
    pltpu.make_async_copy(uT_h.at[:, pl.ds(0, BPW)], ubuf, sem).wait()
    pltpu.make_async_copy(mT_h.at[:, pl.ds(0, BPW)], mbuf, sem).wait()

    def group(g, _):
        rows = g * 16 + lax.iota(jnp.int32, 16)
        base = g * 16
        giv = plsc.load_gather(gidx_v, [rows])
        yiv = plsc.load_gather(yidx_v, [rows])

        def col(c, carry):
            s_um, s_uu, s_mm = carry
            cv = jnp.broadcast_to(c, (16,))
            u = ubuf[c, pl.ds(base, 16)]
            mv = mbuf[c, pl.ds(base, 16)]
            gv = plsc.load_gather(gtbl, [cv, giv])
            yv = plsc.load_gather(ytbl, [cv, yiv])
            m = mv + gv + yv
            return (s_um + u * m, s_uu + u * u, s_mm + m * m)

        zeros = jnp.zeros((16,), jnp.float32)
        s_um, s_uu, s_mm = lax.fori_loop(
            0, D, col, (zeros, zeros, zeros), unroll=8)

        inv = _rsqrt(jnp.maximum(s_uu, 1e-16)) * _rsqrt(jnp.maximum(s_mm, 1e-16))
        ub = plsc.load_gather(ub_v, [rows])
        mb = plsc.load_gather(mb_v, [rows])
        pred = s_um * inv * 2.5 + 2.75 + ub + mb
        plsc.store_scatter(out_v, [rows], pred)
        return 0

    lax.fori_loop(0, NG, group, 0)

    base = pl.multiple_of(wid * BPW, BPW)
    pltpu.sync_copy(out_v, out_h.at[pl.ds(base, BPW)])


def kernel(user_idx, movie_idx, genre_idx, year_idx,
           user_embeds, movie_embeds, genre_embeds, year_embeds,
           user_biases, movie_biases):
    mesh = plsc.VectorSubcoreMesh(core_axis_name="c", subcore_axis_name="s",
                                  num_cores=NC, num_subcores=NS)
    f32 = jnp.float32
    i32 = jnp.int32
    k = pl.kernel(
        _body,
        out_type=jax.ShapeDtypeStruct((B,), f32),
        mesh=mesh,
        compiler_params=pltpu.CompilerParams(needs_layout_passes=False,
                                             use_tc_tiling_on_sc=False),
        scratch_types=[
            pltpu.VMEM((NCHUNK, CHUNK), i32),
            pltpu.VMEM((NCHUNK, CHUNK), i32),
            pltpu.VMEM((BPW,), i32),
            pltpu.VMEM((BPW,), i32),
            pltpu.VMEM((D, BPW), f32),
            pltpu.VMEM((D, BPW), f32),
            pltpu.VMEM((D, 20), f32),
            pltpu.VMEM((D, 100), f32),
            pltpu.VMEM((BPW,), f32),
            pltpu.VMEM((BPW,), f32),
            pltpu.VMEM((BPW,), f32),
            pltpu.SemaphoreType.DMA,
        ],
    )
    uLin = _detile(user_embeds.T, user_embeds[U_FULL:].T,
                   U_NFULL, U_REM, U_TAIL, U_S)
    mLin = _detile(movie_embeds.T, movie_embeds[M_FULL_C:].T,
                   M_NFULL, M_REM, M_TAIL, M_S)
    return k(uLin.reshape(D, U_S), mLin.reshape(D, M_S),
             genre_embeds.T, year_embeds.T,
             user_idx.astype(i32).reshape(NW, NCHUNK, CHUNK),
             movie_idx.astype(i32).reshape(NW, NCHUNK, CHUNK),
             genre_idx.astype(i32).reshape(NW, BPW),
             year_idx.astype(i32).reshape(NW, BPW),
             user_biases.T, movie_biases.T)


# speedup vs baseline: 14.5587x; 1.0617x over previous
import jax
import jax.numpy as jnp
from jax import lax
from jax.experimental import pallas as pl
from jax.experimental.pallas import tpu as pltpu
from jax.experimental.pallas import tpu_sc as plsc

B = 16384
D = 64
NC = 2
NS = 16
NW = NC * NS
BPW = B // NW
NCHUNK = 4
CHUNK = BPW // NCHUNK
NG = BPW // 16

U_CHK = 131072
M_CHK = 65536

U_ROWS, U_FULL, U_TAIL = 1000000, 999936, 64
U_NFULL = U_FULL // U_CHK
U_REM = U_FULL - U_NFULL * U_CHK
U_S = 1000448

M_ROWS, M_FULL_C, M_TAIL = 100000, 99968, 32
M_NFULL = M_FULL_C // M_CHK
M_REM = M_FULL_C - M_NFULL * M_CHK
M_S = 100352


def _detile_body(chk, rem, tail_w, stride, src_ref, tail_ref, out_ref,
                 buf, tbuf, ebuf, sem_in, sem_out):
    a = pl.program_id(0)
    k = pl.program_id(1)
    nfull = pl.num_programs(1) - 1
    edge = ebuf.shape[1]
    CHK = chk

    def start_main(kk, b):
        pltpu.make_async_copy(
            src_ref.at[pl.ds(a * 8, 8), pl.ds(kk * CHK, CHK)],
            buf.at[b], sem_in).start()

    @pl.when(k == 0)
    def _prime():
        start_main(0, 0)
        if nfull > 1:
            start_main(1, 1)

    @pl.when(k >= 1)
    def _drain_prev():
        for i in range(8):
            pltpu.make_async_copy(buf.at[0].at[i], out_ref.at[pl.ds(0, CHK)],
                                  sem_out).wait()

    @pl.when(k + 2 < nfull)
    def _prefetch_main():
        start_main(k + 2, (k + 2) % 3)

    @pl.when(k + 1 == nfull)
    def _prefetch_edge():
        pltpu.make_async_copy(
            src_ref.at[pl.ds(a * 8, 8), pl.ds(nfull * CHK, rem)],
            ebuf.at[:, pl.ds(0, rem)], sem_in).start()
        pltpu.make_async_copy(tail_ref.at[pl.ds(a * 8, 8)], tbuf,
                              sem_in).start()

    @pl.when(k < nfull)
    def _main():
        b = k % 3
        pltpu.make_async_copy(
            src_ref.at[pl.ds(a * 8, 8), pl.ds(k * CHK, CHK)],
            buf.at[b], sem_in).wait()
        for i in range(8):
            dst = pl.ds((a * 8 + i) * stride + k * CHK, CHK)
            pltpu.make_async_copy(buf.at[b].at[i], out_ref.at[dst],
                                  sem_out).start()

    @pl.when(k == nfull)
    def _edge():
        pltpu.make_async_copy(
            src_ref.at[pl.ds(a * 8, 8), pl.ds(nfull * CHK, rem)],
            ebuf.at[:, pl.ds(0, rem)], sem_in).wait()
        pltpu.make_async_copy(tail_ref.at[pl.ds(a * 8, 8)], tbuf,
                              sem_in).wait()
        ebuf[:, pl.ds(rem, tail_w)] = tbuf[...]
        descs = []
        for i in range(8):
            dst = pl.ds((a * 8 + i) * stride + nfull * CHK, edge)
            descs.append(pltpu.async_copy(ebuf.at[i], out_ref.at[dst],
                                          sem_out))
        for dsc in descs:
            dsc.wait()


def _detile(table_t, tail_t, chk, nfull, rem, tail_w, stride):
    import functools
    body = functools.partial(_detile_body, chk, rem, tail_w, stride)
    CHK = chk
    edge = stride - nfull * CHK
    return pl.pallas_call(
        body,
        grid=(8, nfull + 1),
        in_specs=[pl.BlockSpec(memory_space=pl.ANY),
                  pl.BlockSpec(memory_space=pl.ANY)],
        out_specs=pl.BlockSpec(memory_space=pl.ANY),
        out_shape=jax.ShapeDtypeStruct((D * stride,), jnp.float32),
        scratch_shapes=[
            pltpu.VMEM((3, 8, CHK), jnp.float32),
            pltpu.VMEM((8, tail_w), jnp.float32),
            pltpu.VMEM((8, edge), jnp.float32),
            pltpu.SemaphoreType.DMA,
            pltpu.SemaphoreType.DMA,
        ],
    )(table_t, tail_t)


def _rsqrt(x):
    i = plsc.bitcast(x, jnp.int32)
    i = jnp.int32(0x5F3759DF) - (i >> 1)
    y = plsc.bitcast(i, jnp.float32)
    for _ in range(3):
        y = y * (1.5 - 0.5 * x * y * y)
    return y


def _body(uT_h, mT_h, gT_h, yT_h, uidx_h, midx_h, gidx_h, yidx_h,
          ubias_h, mbias_h,
          out_h,
          uidx_v, midx_v, gidx_v, yidx_v,
          ubuf, mbuf, gtbl, ytbl, ub_v, mb_v, out_v, sem):
    wid = lax.axis_index("s") * NC + lax.axis_index("c")

    pltpu.sync_copy(uidx_h.at[wid], uidx_v)
    pltpu.sync_copy(midx_h.at[wid], midx_v)
    pltpu.sync_copy(gidx_h.at[wid], gidx_v)
    pltpu.sync_copy(yidx_h.at[wid], yidx_v)

    descs = [
        pltpu.async_copy(gT_h, gtbl, sem),
        pltpu.async_copy(yT_h, ytbl, sem),
    ]
    for j in range(NCHUNK):
        dst = pl.ds(j * CHUNK, CHUNK)
        descs.append(pltpu.async_copy(ubias_h.at[0].at[uidx_v.at[j]],
                                      ub_v.at[dst], sem))
        descs.append(pltpu.async_copy(mbias_h.at[0].at[midx_v.at[j]],
                                      mb_v.at[dst], sem))

    def plane(c, _):
        for j in range(NCHUNK):
            dst = pl.ds(j * CHUNK, CHUNK)
            pltpu.async_copy(uT_h.at[c].at[uidx_v.at[j]],
                             ubuf.at[c].at[dst], sem)
            pltpu.async_copy(mT_h.at[c].at[midx_v.at[j]],
                             mbuf.at[c].at[dst], sem)
        return 0

    lax.fori_loop(0, D, plane, 0)

    for dsc in descs:
        dsc.wait()
    pltpu.make_async_copy(uT_h.at[:, pl.ds(0, BPW)], ubuf, sem).wait()
    pltpu.make_async_copy(mT_h.at[:, pl.ds(0, BPW)], mbuf, sem).wait()

    def group(g, _):
        rows = g * 16 + lax.iota(jnp.int32, 16)
        base = g * 16
        giv = plsc.load_gather(gidx_v, [rows])
        yiv = plsc.load_gather(yidx_v, [rows])

        def col(c, carry):
            s_um, s_uu, s_mm = carry
            cv = jnp.broadcast_to(c, (16,))
            u = ubuf[c, pl.ds(base, 16)]
            mv = mbuf[c, pl.ds(base, 16)]
            gv = plsc.load_gather(gtbl, [cv, giv])
            yv = plsc.load_gather(ytbl, [cv, yiv])
            m = mv + gv + yv
            return (s_um + u * m, s_uu + u * u, s_mm + m * m)

        zeros = jnp.zeros((16,), jnp.float32)
        s_um, s_uu, s_mm = lax.fori_loop(
            0, D, col, (zeros, zeros, zeros), unroll=8)

        inv = _rsqrt(jnp.maximum(s_uu, 1e-16)) * _rsqrt(jnp.maximum(s_mm, 1e-16))
        ub = plsc.load_gather(ub_v, [rows])
        mb = plsc.load_gather(mb_v, [rows])
        pred = s_um * inv * 2.5 + 2.75 + ub + mb
        plsc.store_scatter(out_v, [rows], pred)
        return 0

    lax.fori_loop(0, NG, group, 0)

    base = pl.multiple_of(wid * BPW, BPW)
    pltpu.sync_copy(out_v, out_h.at[pl.ds(base, BPW)])


def kernel(user_idx, movie_idx, genre_idx, year_idx,
           user_embeds, movie_embeds, genre_embeds, year_embeds,
           user_biases, movie_biases):
    mesh = plsc.VectorSubcoreMesh(core_axis_name="c", subcore_axis_name="s",
                                  num_cores=NC, num_subcores=NS)
    f32 = jnp.float32
    i32 = jnp.int32
    k = pl.kernel(
        _body,
        out_type=jax.ShapeDtypeStruct((B,), f32),
        mesh=mesh,
        compiler_params=pltpu.CompilerParams(needs_layout_passes=False,
                                             use_tc_tiling_on_sc=False),
        scratch_types=[
            pltpu.VMEM((NCHUNK, CHUNK), i32),
            pltpu.VMEM((NCHUNK, CHUNK), i32),
            pltpu.VMEM((BPW,), i32),
            pltpu.VMEM((BPW,), i32),
            pltpu.VMEM((D, BPW), f32),
            pltpu.VMEM((D, BPW), f32),
            pltpu.VMEM((D, 20), f32),
            pltpu.VMEM((D, 100), f32),
            pltpu.VMEM((BPW,), f32),
            pltpu.VMEM((BPW,), f32),
            pltpu.VMEM((BPW,), f32),
            pltpu.SemaphoreType.DMA,
        ],
    )
    uLin = _detile(user_embeds.T, user_embeds[U_FULL:].T,
                   U_CHK, U_NFULL, U_REM, U_TAIL, U_S)
    mLin = _detile(movie_embeds.T, movie_embeds[M_FULL_C:].T,
                   M_CHK, M_NFULL, M_REM, M_TAIL, M_S)
    return k(uLin.reshape(D, U_S), mLin.reshape(D, M_S),
             genre_embeds.T, year_embeds.T,
             user_idx.astype(i32).reshape(NW, NCHUNK, CHUNK),
             movie_idx.astype(i32).reshape(NW, NCHUNK, CHUNK),
             genre_idx.astype(i32).reshape(NW, BPW),
             year_idx.astype(i32).reshape(NW, BPW),
             user_biases.T, movie_biases.T)
